# Initial kernel scaffold; baseline (speedup 1.0000x reference)
import jax
import jax.numpy as jnp
from jax.experimental import pallas as pl


def kernel(x, pos, edge_attr, frag_levels, adduct_feats, params, edge_index, batch):
    raise NotImplementedError("write your pallas kernel here")


# speedup vs baseline: 2.6809x; 2.6809x over previous
import functools
import jax
import jax.numpy as jnp
from jax import lax
from jax.experimental import pallas as pl
from jax.experimental.pallas import tpu as pltpu
from jax.experimental.pallas import tpu_sc as plsc

F32 = jnp.float32
HD = 128
CP = 16
NC = 2
NS = 16
NW = NC * NS
K = 80


def _silu(v):
    return v * (1.0 / (1.0 + jnp.exp(-v)))


def _sc_gather(hs, ht, coordp, row, col):
    N = hs.shape[0]
    E = row.shape[0]
    per_w = E // NW
    n_chunks = per_w // K
    assert per_w % K == 0 and E % NW == 0

    mesh = plsc.VectorSubcoreMesh(core_axis_name="c", subcore_axis_name="s")

    @functools.partial(
        pl.kernel,
        out_type=(
            jax.ShapeDtypeStruct((E, HD), F32),
            jax.ShapeDtypeStruct((E, HD), F32),
            jax.ShapeDtypeStruct((E, CP), F32),
            jax.ShapeDtypeStruct((E, CP), F32),
        ),
        mesh=mesh,
        compiler_params=pltpu.CompilerParams(use_tc_tiling_on_sc=False),
        scratch_types=[
            pltpu.VMEM((K,), jnp.int32),
            pltpu.VMEM((K,), jnp.int32),
            pltpu.VMEM((K, HD), F32),
            pltpu.VMEM((K, HD), F32),
            pltpu.VMEM((K, CP), F32),
            pltpu.VMEM((K, CP), F32),
            pltpu.SemaphoreType.DMA,
            pltpu.SemaphoreType.DMA,
        ],
    )
    def k(hs_h, ht_h, cp_h, row_h, col_h,
          src_h, tgt_h, crow_h, ccol_h,
          ridx, cidx, sbuf, tbuf, crbuf, ccbuf, gsem, wsem):
        wid = lax.axis_index("s") * NC + lax.axis_index("c")
        base = wid * per_w

        def body(i, _):
            off = base + i * K
            pltpu.sync_copy(row_h.at[pl.ds(off, K)], ridx)
            pltpu.sync_copy(col_h.at[pl.ds(off, K)], cidx)
            g1 = pltpu.async_copy(hs_h.at[ridx], sbuf, gsem)
            g2 = pltpu.async_copy(ht_h.at[cidx], tbuf, gsem)
            g3 = pltpu.async_copy(cp_h.at[ridx], crbuf, gsem)
            g4 = pltpu.async_copy(cp_h.at[cidx], ccbuf, gsem)
            g1.wait(); g2.wait(); g3.wait(); g4.wait()
            w1 = pltpu.async_copy(sbuf, src_h.at[pl.ds(off, K)], wsem)
            w2 = pltpu.async_copy(tbuf, tgt_h.at[pl.ds(off, K)], wsem)
            w3 = pltpu.async_copy(crbuf, crow_h.at[pl.ds(off, K)], wsem)
            w4 = pltpu.async_copy(ccbuf, ccol_h.at[pl.ds(off, K)], wsem)
            w1.wait(); w2.wait(); w3.wait(); w4.wait()
            return 0

        lax.fori_loop(0, n_chunks, body, 0, unroll=False)

    return k(hs, ht, coordp, row, col)


def _sc_scatter(m_e, t_e, row, zeros_m, zeros_t):
    E = row.shape[0]
    N = zeros_m.shape[0]
    per_w = E // NW
    n_chunks = per_w // K

    mesh = plsc.VectorSubcoreMesh(core_axis_name="c", subcore_axis_name="s")

    @functools.partial(
        pl.kernel,
        out_type=(
            jax.ShapeDtypeStruct((NC, N, HD), F32),
            jax.ShapeDtypeStruct((NC, N, CP), F32),
        ),
        mesh=mesh,
        compiler_params=pltpu.CompilerParams(use_tc_tiling_on_sc=False),
        scratch_types=[
            pltpu.VMEM((K,), jnp.int32),
            pltpu.VMEM((K, HD), F32),
            pltpu.VMEM((K, CP), F32),
            pltpu.VMEM_SHARED((N, HD), F32),
            pltpu.VMEM_SHARED((N, CP), F32),
            pltpu.SemaphoreType.DMA,
        ],
    )
    def k(m_h, t_h, row_h, zm_h, zt_h, outm_h, outt_h,
          idxb, mbuf, tbuf, accm, acct, sem):
        cid = lax.axis_index("c")
        sid = lax.axis_index("s")

        @pl.when(sid == 0)
        def _():
            pltpu.sync_copy(zm_h, accm)
            pltpu.sync_copy(zt_h, acct)

        plsc.subcore_barrier()

        wid = sid * NC + cid
        base = wid * per_w

        def body(i, _):
            off = base + i * K
            pltpu.sync_copy(row_h.at[pl.ds(off, K)], idxb)
            c1 = pltpu.async_copy(m_h.at[pl.ds(off, K)], mbuf, sem)
            c2 = pltpu.async_copy(t_h.at[pl.ds(off, K)], tbuf, sem)
            c1.wait(); c2.wait()
            pltpu.sync_copy(mbuf, accm.at[idxb], add=True)
            pltpu.sync_copy(tbuf, acct.at[idxb], add=True)
            return 0

        lax.fori_loop(0, n_chunks, body, 0, unroll=False)
        plsc.subcore_barrier()

        @pl.when(sid == 0)
        def _():
            pltpu.sync_copy(accm, outm_h.at[cid])
            pltpu.sync_copy(acct, outt_h.at[cid])

    return k(m_e, t_e, row, zeros_m, zeros_t)


def _tc_init(x, wemb, bemb, ws0, wt0):
    N = x.shape[0]

    def body(x_ref, we_ref, be_ref, ws_ref, wt_ref, h_ref, hs_ref, ht_ref):
        h = jnp.dot(x_ref[...], we_ref[...], preferred_element_type=F32) + be_ref[...]
        h_ref[...] = h
        hs_ref[...] = jnp.dot(h, ws_ref[...], preferred_element_type=F32)
        ht_ref[...] = jnp.dot(h, wt_ref[...], preferred_element_type=F32)

    return pl.pallas_call(
        body,
        out_shape=(
            jax.ShapeDtypeStruct((N, HD), F32),
            jax.ShapeDtypeStruct((N, HD), F32),
            jax.ShapeDtypeStruct((N, HD), F32),
        ),
    )(x, wemb, bemb, ws0, wt0)


def _tc_edge(srcp, tgtp, crow, ccol, edge_attr, we, b1, wr, w2, b2, cw1, cb1, cw2t):
    E = srcp.shape[0]
    BE = 2000
    grid = E // BE
    DE = edge_attr.shape[1]

    def body(src_ref, tgt_ref, cr_ref, cc_ref, ea_ref,
             we_ref, b1_ref, wr_ref, w2_ref, b2_ref, cw1_ref, cb1_ref, cw2_ref,
             m_ref, t_ref):
        cdiff = cr_ref[...] - cc_ref[...]
        radial = jnp.sum(cdiff * cdiff, axis=1, keepdims=True)
        pre = (src_ref[...] + tgt_ref[...]
               + jnp.dot(ea_ref[...], we_ref[...], preferred_element_type=F32)
               + radial * wr_ref[...] + b1_ref[...])
        m1 = _silu(pre)
        m = _silu(jnp.dot(m1, w2_ref[...], preferred_element_type=F32) + b2_ref[...])
        t = _silu(jnp.dot(m, cw1_ref[...], preferred_element_type=F32) + cb1_ref[...])
        c = jnp.sum(t * cw2_ref[...], axis=1, keepdims=True)
        lane = lax.broadcasted_iota(jnp.int32, (BE, CP), 1)
        onec = jnp.where(lane == CP - 1, 1.0, 0.0)
        m_ref[...] = m
        t_ref[...] = cdiff * c + onec

    eb = lambda w: pl.BlockSpec((BE, w), lambda i: (i, 0))
    full = lambda s: pl.BlockSpec(s, lambda i: (0, 0))
    return pl.pallas_call(
        body,
        grid=(grid,),
        in_specs=[
            eb(HD), eb(HD), eb(CP), eb(CP), eb(DE),
            full((DE, HD)), full((1, HD)), full((1, HD)),
            full((HD, HD)), full((1, HD)),
            full((HD, HD)), full((1, HD)), full((1, HD)),
        ],
        out_specs=(eb(HD), eb(CP)),
        out_shape=(
            jax.ShapeDtypeStruct((E, HD), F32),
            jax.ShapeDtypeStruct((E, CP), F32),
        ),
    )(srcp, tgtp, crow, ccol, edge_attr, we, b1, wr, w2, b2, cw1, cb1, cw2t)


def _tc_node(h, pm0, pm1, pc0, pc1, cp, nw1a, nw1b, nb1, nw2, nb2, ws, wt):
    N = h.shape[0]

    def body(h_ref, pm0_ref, pm1_ref, pc0_ref, pc1_ref, cp_ref,
             nw1a_ref, nw1b_ref, nb1_ref, nw2_ref, nb2_ref, ws_ref, wt_ref,
             h_out, cp_out, hs_out, ht_out):
        magg = pm0_ref[...] + pm1_ref[...]
        cagg = pc0_ref[...] + pc1_ref[...]
        cnt = cagg[:, CP - 1:CP]
        inv = 1.0 / jnp.maximum(cnt, 1.0)
        lane = lax.broadcasted_iota(jnp.int32, (N, CP), 1)
        cmask = jnp.where(lane < 3, 1.0, 0.0)
        cp_out[...] = cp_ref[...] + cagg * cmask * inv
        h = h_ref[...]
        o = _silu(jnp.dot(h, nw1a_ref[...], preferred_element_type=F32)
                  + jnp.dot(magg, nw1b_ref[...], preferred_element_type=F32)
                  + nb1_ref[...])
        hn = h + jnp.dot(o, nw2_ref[...], preferred_element_type=F32) + nb2_ref[...]
        h_out[...] = hn
        hs_out[...] = jnp.dot(hn, ws_ref[...], preferred_element_type=F32)
        ht_out[...] = jnp.dot(hn, wt_ref[...], preferred_element_type=F32)

    return pl.pallas_call(
        body,
        out_shape=(
            jax.ShapeDtypeStruct((N, HD), F32),
            jax.ShapeDtypeStruct((N, CP), F32),
            jax.ShapeDtypeStruct((N, HD), F32),
            jax.ShapeDtypeStruct((N, HD), F32),
        ),
    )(h, pm0, pm1, pc0, pc1, cp, nw1a, nw1b, nb1, nw2, nb2, ws, wt)


def _tc_node_final(h, pm0, pm1, nw1a, nw1b, nb1, nw2, nb2, wout, bout):
    N = h.shape[0]

    def body(h_ref, pm0_ref, pm1_ref,
             nw1a_ref, nw1b_ref, nb1_ref, nw2_ref, nb2_ref, wo_ref, bo_ref,
             hout_ref):
        magg = pm0_ref[...] + pm1_ref[...]
        h = h_ref[...]
        o = _silu(jnp.dot(h, nw1a_ref[...], preferred_element_type=F32)
                  + jnp.dot(magg, nw1b_ref[...], preferred_element_type=F32)
                  + nb1_ref[...])
        hn = h + jnp.dot(o, nw2_ref[...], preferred_element_type=F32) + nb2_ref[...]
        hout_ref[...] = jnp.dot(hn, wo_ref[...], preferred_element_type=F32) + bo_ref[...]

    return pl.pallas_call(
        body,
        out_shape=jax.ShapeDtypeStruct((N, HD), F32),
    )(h, pm0, pm1, nw1a, nw1b, nb1, nw2, nb2, wout, bout)


def _tc_head(hout, batch2d, fl, af, rw1, rb1, rw2, rb2, hw, hb, n_graphs):
    N = hout.shape[0]
    P = hb.shape[1]
    GF = fl.shape[1]

    def body(h_ref, b_ref, fl_ref, af_ref,
             rw1_ref, rb1_ref, rw2_ref, rb2_ref, hw_ref, hb_ref, out_ref,
             pooled_ref):
        h = h_ref[...]
        b = b_ref[...]

        def gbody(g, _):
            masked = jnp.where(b == g, h, -3e38)
            mx = jnp.max(masked, axis=0, keepdims=True)
            pooled_ref[pl.ds(g, 1), :] = mx
            return 0

        lax.fori_loop(0, n_graphs, gbody, 0)
        z = jnp.concatenate([pooled_ref[...], fl_ref[...], af_ref[...]], axis=1)
        r = _silu(jnp.dot(z, rw1_ref[...], preferred_element_type=F32) + rb1_ref[...])
        r = jnp.dot(r, rw2_ref[...], preferred_element_type=F32) + rb2_ref[...]
        z = z + r
        out_ref[...] = jnp.dot(z, hw_ref[...], preferred_element_type=F32) + hb_ref[...]

    return pl.pallas_call(
        body,
        out_shape=jax.ShapeDtypeStruct((n_graphs, P), F32),
        scratch_shapes=[pltpu.VMEM((n_graphs, HD), F32)],
    )(hout, batch2d, fl, af, rw1, rb1, rw2, rb2, hw, hb)


def kernel(x, pos, edge_attr, frag_levels, adduct_feats, params, edge_index, batch):
    N = x.shape[0]
    E = edge_index.shape[1]
    row = edge_index[0]
    col = edge_index[1]
    n_graphs = 64
    r2 = lambda b: b.reshape(1, -1)

    coordp = jnp.pad(pos, ((0, 0), (0, CP - pos.shape[1])))
    zeros_m = jnp.zeros((N, HD), F32)
    zeros_t = jnp.zeros((N, CP), F32)

    layers = params['layers']
    Ws = [lp['edge_w1'][:HD] for lp in layers]
    Wt = [lp['edge_w1'][HD:2 * HD] for lp in layers]
    wr = [lp['edge_w1'][2 * HD:2 * HD + 1] for lp in layers]
    We = [lp['edge_w1'][2 * HD + 1:] for lp in layers]
    nw1a = [lp['node_w1'][:HD] for lp in layers]
    nw1b = [lp['node_w1'][HD:] for lp in layers]

    h, hs, ht = _tc_init(x, params['emb_in_w'], r2(params['emb_in_b']), Ws[0], Wt[0])
    cp = coordp

    for l, lp in enumerate(layers):
        srcp, tgtp, crow, ccol = _sc_gather(hs, ht, cp, row, col)
        m_e, t_e = _tc_edge(srcp, tgtp, crow, ccol, edge_attr,
                            We[l], r2(lp['edge_b1']), wr[l],
                            lp['edge_w2'], r2(lp['edge_b2']),
                            lp['coord_w1'], r2(lp['coord_b1']),
                            lp['coord_w2'].reshape(1, HD))
        pm, pc = _sc_scatter(m_e, t_e, row, zeros_m, zeros_t)
        if l + 1 < len(layers):
            h, cp, hs, ht = _tc_node(h, pm[0], pm[1], pc[0], pc[1], cp,
                                     nw1a[l], nw1b[l], r2(lp['node_b1']),
                                     lp['node_w2'], r2(lp['node_b2']),
                                     Ws[l + 1], Wt[l + 1])
        else:
            hout = _tc_node_final(h, pm[0], pm[1],
                                  nw1a[l], nw1b[l], r2(lp['node_b1']),
                                  lp['node_w2'], r2(lp['node_b2']),
                                  params['emb_out_w'], r2(params['emb_out_b']))

    fl = frag_levels.reshape(n_graphs, -1)
    af = adduct_feats.reshape(n_graphs, -1)
    out = _tc_head(hout, batch.reshape(N, 1), fl, af,
                   params['res_w1'], r2(params['res_b1']),
                   params['res_w2'], r2(params['res_b2']),
                   params['head_w'], r2(params['head_b']), n_graphs)
    return out
